# Initial kernel scaffold; baseline (speedup 1.0000x reference)
import jax
import jax.numpy as jnp
from jax.experimental import pallas as pl


def kernel(logit_previous, side_information, v, b, weights, boolean_converter):
    raise NotImplementedError("write your pallas kernel here")


# speedup vs baseline: 5.4258x; 5.4258x over previous
import functools

import jax
import jax.numpy as jnp
from jax import lax
from jax.experimental import pallas as pl
from jax.experimental.pallas import tpu as pltpu

INPUT_DIM = 2048
SIDE_DIM = 2048
CONTEXT_DIM = 4
NUM_CTX = 2 ** CONTEXT_DIM
BATCH = 4096
BB = 512


def _tc_body(side_ref, logit_ref, v_ref, b_ref, w_ref, bc_ref, out_ref):
    proj = jnp.dot(v_ref[...], side_ref[...],
                   preferred_element_type=jnp.float32)
    bits = (proj > b_ref[...]).astype(jnp.float32)
    ctx = jnp.sum(bits * bc_ref[...], axis=0).astype(jnp.int32)
    dots = jnp.dot(w_ref[...], logit_ref[...],
                   preferred_element_type=jnp.float32)
    row_ids = lax.broadcasted_iota(jnp.int32, (NUM_CTX, dots.shape[1]), 0)
    onehot = (row_ids == ctx[None, :]).astype(jnp.float32)
    out_ref[0, :] = jnp.sum(onehot * dots, axis=0)


def kernel(logit_previous, side_information, v, b, weights, boolean_converter):
    grid = BATCH // BB
    out = pl.pallas_call(
        _tc_body,
        grid=(grid,),
        in_specs=[
            pl.BlockSpec((SIDE_DIM, BB), lambda i: (0, i)),
            pl.BlockSpec((INPUT_DIM, BB), lambda i: (0, i)),
            pl.BlockSpec((CONTEXT_DIM, SIDE_DIM), lambda i: (0, 0)),
            pl.BlockSpec((CONTEXT_DIM, 1), lambda i: (0, 0)),
            pl.BlockSpec((NUM_CTX, INPUT_DIM), lambda i: (0, 0)),
            pl.BlockSpec((CONTEXT_DIM, 1), lambda i: (0, 0)),
        ],
        out_specs=pl.BlockSpec((1, BB), lambda i: (0, i)),
        out_shape=jax.ShapeDtypeStruct((1, BATCH), jnp.float32),
    )(side_information, logit_previous, v, b, weights, boolean_converter)
    return out.reshape(BATCH)
